# baseline (speedup 1.0000x reference)
import functools

import jax
import jax.numpy as jnp
from jax.experimental import pallas as pl

NUM_EXPERTS = 64
TOP_K = 8
TB = 512
NS = 2


def _topk8(vals, iota):
    scores = []
    idxs = []
    for _ in range(TOP_K):
        m = jnp.max(vals, axis=-1, keepdims=True)
        i = jnp.argmax(vals, axis=-1, keepdims=True).astype(jnp.int32)
        scores.append(m)
        idxs.append(i)
        vals = jnp.where(iota == i, -jnp.inf, vals)
    return jnp.concatenate(scores, axis=-1), jnp.concatenate(idxs, axis=-1)


def _router_block(x1_ref, x2_ref, wt_ref, s1_ref, i1_ref, s2_ref, i2_ref):
    iota = jax.lax.broadcasted_iota(jnp.int32, (TB, NUM_EXPERTS), 1)
    l1 = jnp.dot(x1_ref[...], wt_ref[...], preferred_element_type=jnp.float32)
    l2 = jnp.dot(x2_ref[...], wt_ref[...], preferred_element_type=jnp.float32)
    s, ix = _topk8(l1, iota)
    s1_ref[...] = s
    i1_ref[...] = ix
    s, ix = _topk8(l2, iota)
    s2_ref[...] = s
    i2_ref[...] = ix


@functools.partial(jax.jit, static_argnames=())
def kernel(hidden_states, W):
    tokens, hidden = hidden_states.shape
    part = tokens // NS
    nb = part // TB
    wt = W.T
    outs = pl.pallas_call(
        _router_block,
        grid=(nb,),
        in_specs=[
            pl.BlockSpec((TB, hidden), lambda i: (i, 0)),
            pl.BlockSpec((TB, hidden), lambda i: (i + nb, 0)),
            pl.BlockSpec((hidden, NUM_EXPERTS), lambda i: (0, 0)),
        ],
        out_specs=[pl.BlockSpec((TB, TOP_K), lambda i: (i, 0))
                   for _ in range(4)],
        out_shape=[
            jax.ShapeDtypeStruct((part, TOP_K), jnp.float32),
            jax.ShapeDtypeStruct((part, TOP_K), jnp.int32),
            jax.ShapeDtypeStruct((part, TOP_K), jnp.float32),
            jax.ShapeDtypeStruct((part, TOP_K), jnp.int32),
        ],
    )(hidden_states, hidden_states, wt)
    scores = jnp.concatenate([outs[0], outs[2]])
    idx = jnp.concatenate([outs[1], outs[3]])
    return scores, idx


# speedup vs baseline: 1.0665x; 1.0665x over previous
import functools

import jax
import jax.numpy as jnp
from jax.experimental import pallas as pl
from jax.experimental.pallas import tpu as pltpu

NUM_EXPERTS = 64
TOP_K = 8
TB = 1024
NS = 2


def _topk8(vals, iota):
    scores = []
    idxs = []
    for _ in range(TOP_K):
        m = jnp.max(vals, axis=-1, keepdims=True)
        i = jnp.argmax(vals, axis=-1, keepdims=True).astype(jnp.int32)
        scores.append(m)
        idxs.append(i)
        vals = jnp.where(iota == i, -jnp.inf, vals)
    return jnp.concatenate(scores, axis=-1), jnp.concatenate(idxs, axis=-1)


def _router_block(x1_ref, x2_ref, wt_ref, s1_ref, i1_ref, s2_ref, i2_ref,
                  scr_ref):
    i = pl.program_id(0)
    slot = jax.lax.rem(i, 2)
    iota = jax.lax.broadcasted_iota(jnp.int32, (TB, NUM_EXPERTS), 1)

    s, ix = _topk8(scr_ref[1 - slot, 0], iota)
    s1_ref[...] = s
    i1_ref[...] = ix
    s, ix = _topk8(scr_ref[1 - slot, 1], iota)
    s2_ref[...] = s
    i2_ref[...] = ix

    scr_ref[slot, 0] = jnp.dot(
        x1_ref[...], wt_ref[...], preferred_element_type=jnp.float32)
    scr_ref[slot, 1] = jnp.dot(
        x2_ref[...], wt_ref[...], preferred_element_type=jnp.float32)


@functools.partial(jax.jit, static_argnames=())
def kernel(hidden_states, W):
    tokens, hidden = hidden_states.shape
    part = tokens // NS
    nb = part // TB
    wt = W.T
    outs = pl.pallas_call(
        _router_block,
        grid=(nb + 1,),
        in_specs=[
            pl.BlockSpec((TB, hidden), lambda i: (jnp.minimum(i, nb - 1), 0)),
            pl.BlockSpec((TB, hidden),
                         lambda i: (jnp.minimum(i, nb - 1) + nb, 0)),
            pl.BlockSpec((hidden, NUM_EXPERTS), lambda i: (0, 0)),
        ],
        out_specs=[pl.BlockSpec((TB, TOP_K), lambda i: (jnp.maximum(i - 1, 0), 0))
                   for _ in range(4)],
        out_shape=[
            jax.ShapeDtypeStruct((part, TOP_K), jnp.float32),
            jax.ShapeDtypeStruct((part, TOP_K), jnp.int32),
            jax.ShapeDtypeStruct((part, TOP_K), jnp.float32),
            jax.ShapeDtypeStruct((part, TOP_K), jnp.int32),
        ],
        scratch_shapes=[pltpu.VMEM((2, NS, TB, NUM_EXPERTS), jnp.float32)],
    )(hidden_states, hidden_states, wt)
    scores = jnp.concatenate([outs[0], outs[2]])
    idx = jnp.concatenate([outs[1], outs[3]])
    return scores, idx
